# Initial kernel scaffold; baseline (speedup 1.0000x reference)
import functools

import jax
import jax.numpy as jnp
from jax import lax
from jax.experimental import pallas as pl
from jax.experimental.pallas import tpu as pltpu
from jax.experimental.pallas import tpu_sc as plsc

CHUNK = 1024


def _gather_rows(flat_idx, table):
    N = flat_idx.shape[0]
    V, D = table.shape
    info = plsc.get_sparse_core_info()
    nw = info.num_cores * info.num_subcores
    per_w = N // nw
    chunk = min(CHUNK, per_w)
    n_chunks = per_w // chunk
    mesh = plsc.VectorSubcoreMesh(core_axis_name="c", subcore_axis_name="s")

    @functools.partial(
        pl.kernel,
        mesh=mesh,
        out_type=jax.ShapeDtypeStruct((N, D), jnp.float32),
        scratch_types=[
            pltpu.VMEM((chunk,), jnp.int32),
            pltpu.VMEM((chunk, D), jnp.float32),
            pltpu.SemaphoreType.DMA,
        ],
    )
    def k(idx_hbm, table_hbm, out_hbm, idx_v, rows_v, sem):
        wid = lax.axis_index("s") * info.num_cores + lax.axis_index("c")
        base = wid * per_w

        def body(i, carry):
            off = base + i * chunk
            pltpu.sync_copy(idx_hbm.at[pl.ds(off, chunk)], idx_v)
            pltpu.async_copy(table_hbm.at[idx_v], rows_v, sem).wait()
            pltpu.sync_copy(rows_v, out_hbm.at[pl.ds(off, chunk)])
            return carry

        lax.fori_loop(0, n_chunks, body, 0)

    return k(flat_idx, table)


def kernel(x, table):
    B, L = x.shape
    V, D = table.shape
    flat = x.reshape(B * L).astype(jnp.int32)
    out = _gather_rows(flat, table)
    return out.reshape(B, L, D)


# speedup vs baseline: 1.8444x; 1.8444x over previous
import functools

import jax
import jax.numpy as jnp
from jax import lax
from jax.experimental import pallas as pl
from jax.experimental.pallas import tpu as pltpu
from jax.experimental.pallas import tpu_sc as plsc

CHUNK = 1024


def _gather_rows(flat_idx, table):
    N = flat_idx.shape[0]
    V, D = table.shape
    info = plsc.get_sparse_core_info()
    nw = info.num_cores * info.num_subcores
    per_w = N // nw
    chunk = min(CHUNK, per_w)
    n_chunks = per_w // chunk
    mesh = plsc.VectorSubcoreMesh(core_axis_name="c", subcore_axis_name="s")

    @functools.partial(
        pl.kernel,
        mesh=mesh,
        out_type=jax.ShapeDtypeStruct((N, D), jnp.float32),
        scratch_types=[
            pltpu.VMEM((chunk,), jnp.int32),
            pltpu.VMEM((chunk, D), jnp.float32),
            pltpu.SemaphoreType.DMA,
        ],
        compiler_params=pltpu.CompilerParams(use_tc_tiling_on_sc=False),
    )
    def k(idx_hbm, table_hbm, out_hbm, idx_v, rows_v, sem):
        wid = lax.axis_index("s") * info.num_cores + lax.axis_index("c")
        base = wid * per_w

        def body(i, carry):
            off = base + i * chunk
            pltpu.sync_copy(idx_hbm.at[pl.ds(off, chunk)], idx_v)
            pltpu.async_copy(table_hbm.at[idx_v], rows_v, sem).wait()
            pltpu.sync_copy(rows_v, out_hbm.at[pl.ds(off, chunk)])
            return carry

        lax.fori_loop(0, n_chunks, body, 0)

    return k(flat_idx, table)


def kernel(x, table):
    B, L = x.shape
    V, D = table.shape
    flat = x.reshape(B * L).astype(jnp.int32)
    out = _gather_rows(flat, table)
    return out.reshape(B, L, D)


# speedup vs baseline: 1.8768x; 1.0175x over previous
import functools

import jax
import jax.numpy as jnp
from jax import lax
from jax.experimental import pallas as pl
from jax.experimental.pallas import tpu as pltpu
from jax.experimental.pallas import tpu_sc as plsc

CHUNK = 512


def _gather_rows(flat_idx, table):
    N = flat_idx.shape[0]
    V, D = table.shape
    info = plsc.get_sparse_core_info()
    nw = info.num_cores * info.num_subcores
    per_w = N // nw
    chunk = min(CHUNK, per_w)
    n_chunks = per_w // chunk
    assert n_chunks % 2 == 0 and n_chunks >= 4
    mesh = plsc.VectorSubcoreMesh(core_axis_name="c", subcore_axis_name="s")

    @functools.partial(
        pl.kernel,
        mesh=mesh,
        out_type=jax.ShapeDtypeStruct((N, D), jnp.float32),
        scratch_types=[
            pltpu.VMEM((per_w,), jnp.int32),
            pltpu.VMEM((chunk, D), jnp.float32),
            pltpu.VMEM((chunk, D), jnp.float32),
            pltpu.SemaphoreType.DMA,
            pltpu.SemaphoreType.DMA,
            pltpu.SemaphoreType.DMA,
            pltpu.SemaphoreType.DMA,
        ],
        compiler_params=pltpu.CompilerParams(use_tc_tiling_on_sc=False),
    )
    def k(idx_hbm, table_hbm, out_hbm, idx_v, rows0, rows1, g0, g1, s0, s1):
        wid = lax.axis_index("s") * info.num_cores + lax.axis_index("c")
        base = wid * per_w
        rows = (rows0, rows1)
        gsem = (g0, g1)
        ssem = (s0, s1)

        pltpu.sync_copy(idx_hbm.at[pl.ds(base, per_w)], idx_v)

        def gather(i, b):
            pltpu.make_async_copy(
                table_hbm.at[idx_v.at[pl.ds(i * chunk, chunk)]],
                rows[b], gsem[b],
            ).start()

        def store(i, b):
            pltpu.make_async_copy(
                rows[b], out_hbm.at[pl.ds(base + i * chunk, chunk)], ssem[b]
            ).start()

        def wait_gather(b):
            pltpu.make_async_copy(
                table_hbm.at[idx_v.at[pl.ds(0, chunk)]], rows[b], gsem[b]
            ).wait()

        def wait_store(b):
            pltpu.make_async_copy(rows[b], out_hbm.at[pl.ds(0, chunk)], ssem[b]).wait()

        gather(0, 0)
        gather(1, 1)

        npairs = n_chunks // 2

        def body(p, carry):
            for b in range(2):
                i = 2 * p + b
                wait_gather(b)
                store(i, b)

                @pl.when(p < npairs - 1)
                def _():
                    wait_store(b)
                    gather(i + 2, b)

            return carry

        lax.fori_loop(0, npairs, body, 0)
        wait_store(0)
        wait_store(1)

    return k(flat_idx, table)


def kernel(x, table):
    B, L = x.shape
    V, D = table.shape
    flat = x.reshape(B * L).astype(jnp.int32)
    out = _gather_rows(flat, table)
    return out.reshape(B, L, D)
